# baseline (speedup 1.0000x reference)
import functools

import jax
import jax.numpy as jnp
from jax import lax
from jax.experimental import pallas as pl
from jax.experimental.pallas import tpu as pltpu
from jax.experimental.pallas import tpu_sc as plsc

BATCH = 16384
EMBED_DIM = 32
NROWS = 1000000
PACK = 4
PROWS = NROWS // PACK
L = 16
NC = 2
NS = 16
NW = NC * NS
BPW = BATCH // NW
CHUNK = 128
NCHUNK = BPW // CHUNK
NBUF = 2
NG = CHUNK // L


def _mesh():
    return plsc.VectorSubcoreMesh(core_axis_name="c", subcore_axis_name="s")


@functools.partial(
    pl.kernel,
    out_type=jax.ShapeDtypeStruct((BATCH,), jnp.float32),
    mesh=_mesh(),
    scratch_types=[
        pltpu.VMEM((BPW,), jnp.int32),
        pltpu.VMEM((BPW,), jnp.int32),
        pltpu.VMEM((BPW,), jnp.int32),
        pltpu.VMEM((BPW,), jnp.int32),
        pltpu.VMEM((BPW,), jnp.int32),
        pltpu.VMEM((BPW,), jnp.int32),
        pltpu.VMEM((NBUF, CHUNK, 128), jnp.float32),
        pltpu.VMEM((NBUF, CHUNK, 128), jnp.float32),
        pltpu.VMEM((2 * EMBED_DIM, L), jnp.float32),
        pltpu.VMEM((L,), jnp.float32),
        pltpu.VMEM((BPW,), jnp.float32),
        pltpu.SemaphoreType.DMA,
        pltpu.SemaphoreType.DMA,
    ],
    compiler_params=pltpu.CompilerParams(needs_layout_passes=False),
)
def _sc_kernel(users_hbm, movies_hbm, ut_hbm, mt_hbm, wb_hbm, bv_hbm, out_hbm,
               uidx_v, midx_v, ush_v, msh_v, uoff_v, moff_v,
               urows_v, mrows_v, wb_v, bv_v, acc_v, usem, msem):
    wid = lax.axis_index("s") * NC + lax.axis_index("c")
    base = wid * BPW

    pltpu.sync_copy(users_hbm.at[pl.ds(base, BPW)], uidx_v)
    pltpu.sync_copy(movies_hbm.at[pl.ds(base, BPW)], midx_v)
    pltpu.sync_copy(wb_hbm, wb_v)
    pltpu.sync_copy(bv_hbm, bv_v)

    def prep(k, _):
        sl = pl.ds(k * L, L)
        uv = uidx_v[sl]
        mv = midx_v[sl]
        ush_v[sl] = uv >> 2
        msh_v[sl] = mv >> 2
        uoff_v[sl] = (uv & 3) * EMBED_DIM
        moff_v[sl] = (mv & 3) * EMBED_DIM
        return 0

    lax.fori_loop(0, BPW // L, prep, 0, unroll=4)

    def fire(j):
        slot = j % NBUF
        sl = pl.ds(j * CHUNK, CHUNK)
        uc = pltpu.async_copy(ut_hbm.at[ush_v.at[sl]], urows_v.at[slot], usem)
        mc = pltpu.async_copy(mt_hbm.at[msh_v.at[sl]], mrows_v.at[slot], msem)
        return uc, mc

    copies = [fire(0), fire(1)]

    bias = bv_v[...]
    lanes = lax.iota(jnp.int32, L)
    slotv = [jnp.full((L,), s, jnp.int32) for s in range(NBUF)]
    ivecs = [lanes + k * L for k in range(NG)]

    for j in range(NCHUNK):
        slot = j % NBUF
        uc, mc = copies[j]
        uc.wait()
        mc.wait()

        uoffs = [uoff_v[pl.ds(j * CHUNK + k * L, L)] for k in range(NG)]
        moffs = [moff_v[pl.ds(j * CHUNK + k * L, L)] for k in range(NG)]

        def col(c, accs, slot=slot, uoffs=uoffs, moffs=moffs):
            wu = wb_v[c, pl.ds(0, L)]
            wm = wb_v[c + EMBED_DIM, pl.ds(0, L)]
            out = []
            for k in range(NG):
                u = plsc.load_gather(urows_v, [slotv[slot], ivecs[k], uoffs[k] + c])
                m = plsc.load_gather(mrows_v, [slotv[slot], ivecs[k], moffs[k] + c])
                out.append(accs[k] + u * wu + m * wm)
            return tuple(out)

        accs = lax.fori_loop(0, EMBED_DIM, col, tuple([bias] * NG))
        for k in range(NG):
            acc_v[pl.ds(j * CHUNK + k * L, L)] = accs[k]
        if j + NBUF < NCHUNK:
            copies.append(fire(j + NBUF))

    pltpu.sync_copy(acc_v, out_hbm.at[pl.ds(base, BPW)])


def kernel(users, movies, user_table, movie_table, W, b):
    ut = user_table.reshape(PROWS, PACK * EMBED_DIM)
    mt = movie_table.reshape(PROWS, PACK * EMBED_DIM)
    w_flat = W.reshape(2 * EMBED_DIM).astype(jnp.float32)
    wb = jnp.broadcast_to(w_flat[:, None], (2 * EMBED_DIM, L))
    bv = jnp.full((L,), b[0], dtype=jnp.float32)
    out = _sc_kernel(users.astype(jnp.int32), movies.astype(jnp.int32),
                     ut, mt, wb, bv)
    return out.reshape(BATCH, 1)
